# Initial kernel scaffold; baseline (speedup 1.0000x reference)
import jax
import jax.numpy as jnp
from jax.experimental import pallas as pl


def kernel(entity_emb, edge_index, edge_type, weight):
    raise NotImplementedError("write your pallas kernel here")


# speedup vs baseline: 1.8231x; 1.8231x over previous
import functools

import jax
import jax.numpy as jnp
from jax import lax
from jax.experimental import pallas as pl
from jax.experimental.pallas import tpu as pltpu
from jax.experimental.pallas import tpu_sc as plsc

N_ENT = 10000
N_EDGE = 320000
CH = 128
N_REL_W = 15

NC, NS = 2, 16
CPC = CH // NC
JPC = CPC // 16
CNTW = 16
ACC = CPC + CNTW
EPT = N_EDGE // NS
CHUNK = 80
NCHUNK = EPT // CHUNK
RPT = (N_ENT // NS) // 8 * 8
REM = N_ENT - NS * RPT


def _sc_edge_body(emb_hbm, tail_hbm, head_hbm, et_hbm, w_hbm, zz_hbm,
                  psum_hbm,
                  rows_v, msg_v, tail_v, head_v, et_v, wbase_v, w_v,
                  sums_sh, gsem):
    c = lax.axis_index("c")
    s = lax.axis_index("s")
    ebase = s * EPT
    nb = s * RPT

    zf = jnp.zeros((16,), jnp.float32)
    of = jnp.ones((16,), jnp.float32)

    @pl.loop(0, CHUNK)
    def _zero_msg(i):
        for j in range(ACC // 16):
            msg_v[i, pl.ds(j * 16, 16)] = zf

    pltpu.sync_copy(w_hbm, w_v)

    pltpu.sync_copy(zz_hbm.at[pl.ds(nb, RPT)], sums_sh.at[pl.ds(nb, RPT)])

    @pl.when(s == NS - 1)
    def _zero_rem():
        pltpu.sync_copy(zz_hbm.at[pl.ds(NS * RPT, REM)],
                        sums_sh.at[pl.ds(NS * RPT, REM)])

    @pl.loop(0, CHUNK)
    def _fill_ones(i):
        msg_v[i, pl.ds(CPC, CNTW)] = of

    plsc.subcore_barrier()

    @pl.loop(0, NCHUNK)
    def _chunk(g):
        base = ebase + g * CHUNK
        pltpu.sync_copy(tail_hbm.at[pl.ds(base, CHUNK)], tail_v)
        pltpu.sync_copy(head_hbm.at[pl.ds(base, CHUNK)], head_v.at[0])
        pltpu.sync_copy(et_hbm.at[pl.ds(base, CHUNK)], et_v)

        pltpu.async_copy(emb_hbm.at[tail_v], rows_v, gsem).wait()

        @pl.loop(0, CHUNK // 16)
        def _wb(k):
            v = et_v[pl.ds(k * 16, 16)]
            wbase_v[pl.ds(k * 16, 16)] = ((v + 14) % N_REL_W) * CH + c * CPC

        coff = c * CPC

        @pl.loop(0, CHUNK // 16)
        def _edge16(k):
            wb16 = wbase_v[pl.ds(k * 16, 16)]
            for l in range(16):
                e = k * 16 + l
                wb = wb16[l]
                for j in range(JPC):
                    wv = w_v[pl.ds(wb + j * 16, 16)]
                    ev = rows_v[e, pl.ds(coff + j * 16, 16)]
                    msg_v[e, pl.ds(j * 16, 16)] = ev * wv

        pltpu.sync_copy(msg_v, sums_sh.at[head_v.at[0]], add=True)

    plsc.subcore_barrier()
    pltpu.sync_copy(sums_sh.at[pl.ds(nb, RPT)], psum_hbm.at[c, pl.ds(nb, RPT)])

    @pl.when(s == NS - 1)
    def _flush_rem():
        pltpu.sync_copy(sums_sh.at[pl.ds(NS * RPT, REM)],
                        psum_hbm.at[c, pl.ds(NS * RPT, REM)])


@functools.cache
def _get_sc_edge():
    return pl.kernel(
        _sc_edge_body,
        mesh=plsc.VectorSubcoreMesh(core_axis_name="c", subcore_axis_name="s",
                                    num_cores=NC, num_subcores=NS),
        compiler_params=pltpu.CompilerParams(use_tc_tiling_on_sc=False),
        out_type=jax.ShapeDtypeStruct((NC, N_ENT, ACC), jnp.float32),
        scratch_types=[
            pltpu.VMEM((CHUNK, CH), jnp.float32),
            pltpu.VMEM((CHUNK, ACC), jnp.float32),
            pltpu.VMEM((CHUNK,), jnp.int32),
            pltpu.VMEM((1, CHUNK), jnp.int32),
            pltpu.VMEM((CHUNK,), jnp.int32),
            pltpu.VMEM((CHUNK,), jnp.int32),
            pltpu.VMEM((N_REL_W * CH,), jnp.float32),
            pltpu.VMEM_SHARED((N_ENT, ACC), jnp.float32),
            pltpu.SemaphoreType.DMA,
        ],
    )


def _sc_edge(*args):
    return _get_sc_edge()(*args)


def _tc_combine_body(ps0_ref, ps1_ref, cnt_ref, res_ref, emb_ref, res_out_ref):
    sums = jnp.concatenate([ps0_ref[...], ps1_ref[...]], axis=-1)
    cnt = cnt_ref[...][:, :1]
    mean = sums / jnp.maximum(cnt, 1.0)
    norm = jnp.sqrt(jnp.sum(mean * mean, axis=1, keepdims=True))
    emb = mean / jnp.maximum(norm, 1e-12)
    emb_ref[...] = emb
    res_out_ref[...] = res_ref[...] + emb


_TC_ROWS = 2000


def _tc_combine(ps0, ps1, cnt, res):
    grid = N_ENT // _TC_ROWS
    h_spec = pl.BlockSpec((_TC_ROWS, CPC), lambda i: (i, 0))
    c_spec = pl.BlockSpec((_TC_ROWS, CNTW), lambda i: (i, 0))
    f_spec = pl.BlockSpec((_TC_ROWS, CH), lambda i: (i, 0))
    return pl.pallas_call(
        _tc_combine_body,
        grid=(grid,),
        in_specs=[h_spec, h_spec, c_spec, f_spec],
        out_specs=[f_spec, f_spec],
        out_shape=(
            jax.ShapeDtypeStruct((N_ENT, CH), jnp.float32),
            jax.ShapeDtypeStruct((N_ENT, CH), jnp.float32),
        ),
    )(ps0, ps1, cnt, res)


def kernel(entity_emb, edge_index, edge_type, weight):
    tail = edge_index[1].astype(jnp.int32)
    head = edge_index[0].astype(jnp.int32)
    et = edge_type.astype(jnp.int32)
    w_flat = weight.reshape(-1)
    zz = jnp.zeros((N_ENT, ACC), jnp.float32)

    def hop(carry, _):
        emb, res = carry
        psum = _sc_edge(emb, tail, head, et, w_flat, zz)
        ps0 = psum[0, :, :CPC]
        ps1 = psum[1, :, :CPC]
        cnt = psum[0, :, CPC:]
        emb_n, res_n = _tc_combine(ps0, ps1, cnt, res)
        return (emb_n, res_n), None

    (_, res2), _ = jax.lax.scan(hop, (entity_emb, entity_emb), None, length=2)
    return res2


# speedup vs baseline: 2.3991x; 1.3159x over previous
import functools

import jax
import jax.numpy as jnp
from jax import lax
from jax.experimental import pallas as pl
from jax.experimental.pallas import tpu as pltpu
from jax.experimental.pallas import tpu_sc as plsc

N_ENT = 10000
N_EDGE = 320000
CH = 128
N_REL_W = 15

NC, NS = 2, 16
CPC = CH // NC
JPC = CPC // 16
CNTW = 16
ACC = CPC + CNTW
EPT = N_EDGE // NS
CHUNK = 80
NCHUNK = EPT // CHUNK
SLAB = 125
NSLAB = NCHUNK // SLAB
RPT = (N_ENT // NS) // 8 * 8
REM = N_ENT - NS * RPT


def _sc_edge_body(emb_hbm, tail_hbm, head_hbm, et_hbm, w_hbm, zz_hbm,
                  psum_hbm,
                  rows_v, msg_v, tail_v, head_v, et_v, wbase_v, w_v,
                  sums_sh, gsem):
    c = lax.axis_index("c")
    s = lax.axis_index("s")
    ebase = s * EPT
    nb = s * RPT

    zf = jnp.zeros((16,), jnp.float32)
    of = jnp.ones((16,), jnp.float32)

    @pl.loop(0, CHUNK)
    def _zero_msg(i):
        for j in range(ACC // 16):
            msg_v[i, pl.ds(j * 16, 16)] = zf

    pltpu.sync_copy(w_hbm, w_v)

    pltpu.sync_copy(zz_hbm.at[pl.ds(nb, RPT)], sums_sh.at[pl.ds(nb, RPT)])

    @pl.when(s == NS - 1)
    def _zero_rem():
        pltpu.sync_copy(zz_hbm.at[pl.ds(NS * RPT, REM)],
                        sums_sh.at[pl.ds(NS * RPT, REM)])

    @pl.loop(0, CHUNK)
    def _fill_ones(i):
        msg_v[i, pl.ds(CPC, CNTW)] = of

    plsc.subcore_barrier()

    coff = c * CPC
    srow = s * NCHUNK

    @pl.loop(0, NSLAB)
    def _slab(b):
        rbase = srow + b * SLAB
        pltpu.sync_copy(tail_hbm.at[pl.ds(rbase, SLAB)], tail_v)
        pltpu.sync_copy(head_hbm.at[pl.ds(rbase, SLAB)], head_v)
        pltpu.sync_copy(et_hbm.at[pl.ds(rbase, SLAB)], et_v)

        @pl.loop(0, SLAB)
        def _chunk(g):
            pltpu.async_copy(emb_hbm.at[tail_v.at[g]], rows_v, gsem).wait()

            @pl.loop(0, CHUNK // 16)
            def _wb(k):
                v = et_v[g, pl.ds(k * 16, 16)]
                wbase_v[pl.ds(k * 16, 16)] = ((v + 14) % N_REL_W) * CH + c * CPC

            @pl.loop(0, CHUNK // 16)
            def _edge16(k):
                wb16 = wbase_v[pl.ds(k * 16, 16)]
                for l in range(16):
                    e = k * 16 + l
                    wb = wb16[l]
                    for j in range(JPC):
                        wv = w_v[pl.ds(wb + j * 16, 16)]
                        ev = rows_v[e, pl.ds(coff + j * 16, 16)]
                        msg_v[e, pl.ds(j * 16, 16)] = ev * wv

            pltpu.sync_copy(msg_v, sums_sh.at[head_v.at[g]], add=True)

    plsc.subcore_barrier()
    pltpu.sync_copy(sums_sh.at[pl.ds(nb, RPT)], psum_hbm.at[c, pl.ds(nb, RPT)])

    @pl.when(s == NS - 1)
    def _flush_rem():
        pltpu.sync_copy(sums_sh.at[pl.ds(NS * RPT, REM)],
                        psum_hbm.at[c, pl.ds(NS * RPT, REM)])


@functools.cache
def _get_sc_edge():
    return pl.kernel(
        _sc_edge_body,
        mesh=plsc.VectorSubcoreMesh(core_axis_name="c", subcore_axis_name="s",
                                    num_cores=NC, num_subcores=NS),
        compiler_params=pltpu.CompilerParams(use_tc_tiling_on_sc=False),
        out_type=jax.ShapeDtypeStruct((NC, N_ENT, ACC), jnp.float32),
        scratch_types=[
            pltpu.VMEM((CHUNK, CH), jnp.float32),
            pltpu.VMEM((CHUNK, ACC), jnp.float32),
            pltpu.VMEM((SLAB, CHUNK), jnp.int32),
            pltpu.VMEM((SLAB, CHUNK), jnp.int32),
            pltpu.VMEM((SLAB, CHUNK), jnp.int32),
            pltpu.VMEM((CHUNK,), jnp.int32),
            pltpu.VMEM((N_REL_W * CH,), jnp.float32),
            pltpu.VMEM_SHARED((N_ENT, ACC), jnp.float32),
            pltpu.SemaphoreType.DMA,
        ],
    )


def _sc_edge(*args):
    return _get_sc_edge()(*args)


def _tc_combine_body(ps0_ref, ps1_ref, cnt_ref, res_ref, emb_ref, res_out_ref):
    sums = jnp.concatenate([ps0_ref[...], ps1_ref[...]], axis=-1)
    cnt = cnt_ref[...][:, :1]
    mean = sums / jnp.maximum(cnt, 1.0)
    norm = jnp.sqrt(jnp.sum(mean * mean, axis=1, keepdims=True))
    emb = mean / jnp.maximum(norm, 1e-12)
    emb_ref[...] = emb
    res_out_ref[...] = res_ref[...] + emb


_TC_ROWS = 2000


def _tc_combine(ps0, ps1, cnt, res):
    grid = N_ENT // _TC_ROWS
    h_spec = pl.BlockSpec((_TC_ROWS, CPC), lambda i: (i, 0))
    c_spec = pl.BlockSpec((_TC_ROWS, CNTW), lambda i: (i, 0))
    f_spec = pl.BlockSpec((_TC_ROWS, CH), lambda i: (i, 0))
    return pl.pallas_call(
        _tc_combine_body,
        grid=(grid,),
        in_specs=[h_spec, h_spec, c_spec, f_spec],
        out_specs=[f_spec, f_spec],
        out_shape=(
            jax.ShapeDtypeStruct((N_ENT, CH), jnp.float32),
            jax.ShapeDtypeStruct((N_ENT, CH), jnp.float32),
        ),
    )(ps0, ps1, cnt, res)


def kernel(entity_emb, edge_index, edge_type, weight):
    tail = edge_index[1].astype(jnp.int32).reshape(N_EDGE // CHUNK, CHUNK)
    head = edge_index[0].astype(jnp.int32).reshape(N_EDGE // CHUNK, CHUNK)
    et = edge_type.astype(jnp.int32).reshape(N_EDGE // CHUNK, CHUNK)
    w_flat = weight.reshape(-1)
    zz = jnp.zeros((N_ENT, ACC), jnp.float32)

    def hop(carry, _):
        emb, res = carry
        psum = _sc_edge(emb, tail, head, et, w_flat, zz)
        ps0 = psum[0, :, :CPC]
        ps1 = psum[1, :, :CPC]
        cnt = psum[0, :, CPC:]
        emb_n, res_n = _tc_combine(ps0, ps1, cnt, res)
        return (emb_n, res_n), None

    (_, res2), _ = jax.lax.scan(hop, (entity_emb, entity_emb), None, length=2)
    return res2


# speedup vs baseline: 3.3370x; 1.3910x over previous
import functools

import jax
import jax.numpy as jnp
from jax import lax
from jax.experimental import pallas as pl
from jax.experimental.pallas import tpu as pltpu
from jax.experimental.pallas import tpu_sc as plsc

N_ENT = 10000
N_EDGE = 320000
CH = 128
N_REL_W = 15

NC, NS = 2, 16
CPC = CH // NC
JPC = CPC // 16
CNTW = 16
ACC = CPC + CNTW
EPT = N_EDGE // NS
CHUNK = 80
NCHUNK = EPT // CHUNK
SLAB = 50
NSLAB = NCHUNK // SLAB
RPT = (N_ENT // NS) // 8 * 8
REM = N_ENT - NS * RPT


def _sc_edge_body(emb_hbm, tail_hbm, head_hbm, et_hbm, w_hbm, zz_hbm,
                  psum_hbm,
                  rows_v, rows_w, msg_v, tail_v, head_v, et_v, wbase_v, w_v,
                  sums_sh, gsem, hsem):
    c = lax.axis_index("c")
    s = lax.axis_index("s")
    ebase = s * EPT
    nb = s * RPT

    zf = jnp.zeros((16,), jnp.float32)
    of = jnp.ones((16,), jnp.float32)

    @pl.loop(0, CHUNK)
    def _zero_msg(i):
        for j in range(ACC // 16):
            msg_v[i, pl.ds(j * 16, 16)] = zf

    pltpu.sync_copy(w_hbm, w_v)

    pltpu.sync_copy(zz_hbm.at[pl.ds(nb, RPT)], sums_sh.at[pl.ds(nb, RPT)])

    @pl.when(s == NS - 1)
    def _zero_rem():
        pltpu.sync_copy(zz_hbm.at[pl.ds(NS * RPT, REM)],
                        sums_sh.at[pl.ds(NS * RPT, REM)])

    @pl.loop(0, CHUNK)
    def _fill_ones(i):
        msg_v[i, pl.ds(CPC, CNTW)] = of

    plsc.subcore_barrier()

    coff = c * CPC
    srow = s * NCHUNK

    def _process(rows_buf, g):
        @pl.loop(0, CHUNK // 16)
        def _wb(k):
            v = et_v[g, pl.ds(k * 16, 16)]
            wbase_v[pl.ds(k * 16, 16)] = ((v + 14) % N_REL_W) * CH + c * CPC

        @pl.loop(0, CHUNK // 16)
        def _edge16(k):
            wb16 = wbase_v[pl.ds(k * 16, 16)]
            for l in range(16):
                e = k * 16 + l
                wb = wb16[l]
                for j in range(JPC):
                    wv = w_v[pl.ds(wb + j * 16, 16)]
                    ev = rows_buf[e, pl.ds(coff + j * 16, 16)]
                    msg_v[e, pl.ds(j * 16, 16)] = ev * wv

        pltpu.sync_copy(msg_v, sums_sh.at[head_v.at[g]], add=True)

    @pl.loop(0, NSLAB)
    def _slab(b):
        rbase = srow + b * SLAB
        pltpu.sync_copy(tail_hbm.at[pl.ds(rbase, SLAB)], tail_v)
        pltpu.sync_copy(head_hbm.at[pl.ds(rbase, SLAB)], head_v)
        pltpu.sync_copy(et_hbm.at[pl.ds(rbase, SLAB)], et_v)

        pltpu.async_copy(emb_hbm.at[tail_v.at[0]], rows_v, gsem)

        @pl.loop(0, SLAB // 2)
        def _pair(t):
            g0 = 2 * t
            pltpu.make_async_copy(emb_hbm.at[tail_v.at[g0]],
                                  rows_v, gsem).wait()
            pltpu.async_copy(emb_hbm.at[tail_v.at[g0 + 1]], rows_w, hsem)
            _process(rows_v, g0)
            pltpu.make_async_copy(emb_hbm.at[tail_v.at[g0 + 1]],
                                  rows_w, hsem).wait()

            @pl.when(t < SLAB // 2 - 1)
            def _pref():
                pltpu.async_copy(emb_hbm.at[tail_v.at[g0 + 2]], rows_v, gsem)

            _process(rows_w, g0 + 1)

    plsc.subcore_barrier()
    pltpu.sync_copy(sums_sh.at[pl.ds(nb, RPT)], psum_hbm.at[c, pl.ds(nb, RPT)])

    @pl.when(s == NS - 1)
    def _flush_rem():
        pltpu.sync_copy(sums_sh.at[pl.ds(NS * RPT, REM)],
                        psum_hbm.at[c, pl.ds(NS * RPT, REM)])


@functools.cache
def _get_sc_edge():
    return pl.kernel(
        _sc_edge_body,
        mesh=plsc.VectorSubcoreMesh(core_axis_name="c", subcore_axis_name="s",
                                    num_cores=NC, num_subcores=NS),
        compiler_params=pltpu.CompilerParams(use_tc_tiling_on_sc=False),
        out_type=jax.ShapeDtypeStruct((NC, N_ENT, ACC), jnp.float32),
        scratch_types=[
            pltpu.VMEM((CHUNK, CH), jnp.float32),
            pltpu.VMEM((CHUNK, CH), jnp.float32),
            pltpu.VMEM((CHUNK, ACC), jnp.float32),
            pltpu.VMEM((SLAB, CHUNK), jnp.int32),
            pltpu.VMEM((SLAB, CHUNK), jnp.int32),
            pltpu.VMEM((SLAB, CHUNK), jnp.int32),
            pltpu.VMEM((CHUNK,), jnp.int32),
            pltpu.VMEM((N_REL_W * CH,), jnp.float32),
            pltpu.VMEM_SHARED((N_ENT, ACC), jnp.float32),
            pltpu.SemaphoreType.DMA,
            pltpu.SemaphoreType.DMA,
        ],
    )


def _sc_edge(*args):
    return _get_sc_edge()(*args)


def _tc_combine_body(ps0_ref, ps1_ref, cnt_ref, res_ref, emb_ref, res_out_ref):
    sums = jnp.concatenate([ps0_ref[...], ps1_ref[...]], axis=-1)
    cnt = cnt_ref[...][:, :1]
    mean = sums / jnp.maximum(cnt, 1.0)
    norm = jnp.sqrt(jnp.sum(mean * mean, axis=1, keepdims=True))
    emb = mean / jnp.maximum(norm, 1e-12)
    emb_ref[...] = emb
    res_out_ref[...] = res_ref[...] + emb


_TC_ROWS = 2000


def _tc_combine(ps0, ps1, cnt, res):
    grid = N_ENT // _TC_ROWS
    h_spec = pl.BlockSpec((_TC_ROWS, CPC), lambda i: (i, 0))
    c_spec = pl.BlockSpec((_TC_ROWS, CNTW), lambda i: (i, 0))
    f_spec = pl.BlockSpec((_TC_ROWS, CH), lambda i: (i, 0))
    return pl.pallas_call(
        _tc_combine_body,
        grid=(grid,),
        in_specs=[h_spec, h_spec, c_spec, f_spec],
        out_specs=[f_spec, f_spec],
        out_shape=(
            jax.ShapeDtypeStruct((N_ENT, CH), jnp.float32),
            jax.ShapeDtypeStruct((N_ENT, CH), jnp.float32),
        ),
    )(ps0, ps1, cnt, res)


def kernel(entity_emb, edge_index, edge_type, weight):
    tail = edge_index[1].astype(jnp.int32).reshape(N_EDGE // CHUNK, CHUNK)
    head = edge_index[0].astype(jnp.int32).reshape(N_EDGE // CHUNK, CHUNK)
    et = edge_type.astype(jnp.int32).reshape(N_EDGE // CHUNK, CHUNK)
    w_flat = weight.reshape(-1)
    zz = jnp.zeros((N_ENT, ACC), jnp.float32)

    def hop(carry, _):
        emb, res = carry
        psum = _sc_edge(emb, tail, head, et, w_flat, zz)
        ps0 = psum[0, :, :CPC]
        ps1 = psum[1, :, :CPC]
        cnt = psum[0, :, CPC:]
        emb_n, res_n = _tc_combine(ps0, ps1, cnt, res)
        return (emb_n, res_n), None

    (_, res2), _ = jax.lax.scan(hop, (entity_emb, entity_emb), None, length=2)
    return res2
